# baseline (speedup 1.0000x reference)
import functools

import jax
import jax.numpy as jnp
from jax import lax
from jax.experimental import pallas as pl
from jax.experimental.pallas import tpu as pltpu
from jax.experimental.pallas import tpu_sc as plsc

N = 4096
ROWS_BLK = 256
N_BLKS = N // ROWS_BLK
NBKT = 64
BKT = N // NBKT
CAP = 96
CCAP = 32 * CAP


def _elu(x):
    return jnp.where(x > 0, x, jnp.exp(x) - 1.0)


def _ln_rows(x, g, b, eps=1e-5):
    m = jnp.mean(x, axis=-1, keepdims=True)
    v = jnp.mean((x - m) ** 2, axis=-1, keepdims=True)
    return (x - m) / jnp.sqrt(v + eps) * g + b


def _pre_body(x_ref, g_ref, b_ref, w1_ref, a_ref, h_ref, s_ref):
    xn = _ln_rows(x_ref[...], g_ref[...], b_ref[...])
    h = jnp.dot(xn, w1_ref[...], preferred_element_type=jnp.float32)
    h_ref[...] = h
    s_ref[...] = jnp.dot(h, a_ref[...], preferred_element_type=jnp.float32)


def _tc_pre(X, g, b, W1, A):
    return pl.pallas_call(
        _pre_body,
        grid=(N_BLKS,),
        in_specs=[
            pl.BlockSpec((ROWS_BLK, 512), lambda i: (i, 0)),
            pl.BlockSpec((512,), lambda i: (0,)),
            pl.BlockSpec((512,), lambda i: (0,)),
            pl.BlockSpec((512, 1024), lambda i: (0, 0)),
            pl.BlockSpec((1024, 128), lambda i: (0, 0)),
        ],
        out_specs=[
            pl.BlockSpec((ROWS_BLK, 1024), lambda i: (i, 0)),
            pl.BlockSpec((ROWS_BLK, 128), lambda i: (i, 0)),
        ],
        out_shape=[
            jax.ShapeDtypeStruct((N, 1024), jnp.float32),
            jax.ShapeDtypeStruct((N, 128), jnp.float32),
        ],
    )(X, g, b, W1, A)


def _mid_body(ms_ref, a_ref, w2_ref, am_ref, h_ref, s_ref, *, heads):
    f = 1024 // heads
    asum = a_ref[...]
    div = jnp.repeat(asum + 1e-16, f, axis=1)
    h1 = _elu(ms_ref[...] / div)
    h = jnp.dot(h1, w2_ref[...], preferred_element_type=jnp.float32)
    h_ref[...] = h
    s_ref[...] = jnp.dot(h, am_ref[...], preferred_element_type=jnp.float32)


def _tc_mid(msum1, asum1, W2, A):
    heads = asum1.shape[-1]
    body = functools.partial(_mid_body, heads=heads)
    return pl.pallas_call(
        body,
        grid=(N_BLKS,),
        in_specs=[
            pl.BlockSpec((ROWS_BLK, 1024), lambda i: (i, 0)),
            pl.BlockSpec((ROWS_BLK, heads), lambda i: (i, 0)),
            pl.BlockSpec((1024, 1024), lambda i: (0, 0)),
            pl.BlockSpec((1024, 128), lambda i: (0, 0)),
        ],
        out_specs=[
            pl.BlockSpec((ROWS_BLK, 1024), lambda i: (i, 0)),
            pl.BlockSpec((ROWS_BLK, 128), lambda i: (i, 0)),
        ],
        out_shape=[
            jax.ShapeDtypeStruct((N, 1024), jnp.float32),
            jax.ShapeDtypeStruct((N, 128), jnp.float32),
        ],
    )(msum1, asum1, W2, A)


def _post_body(ms_ref, a_ref, x_ref, g_ref, b_ref, wc_ref, bc_ref,
               wr_ref, br_ref, o_ref):
    asum = a_ref[...]
    ha = ms_ref[:, :512] / (asum[:, 0:1] + 1e-16)
    hb = ms_ref[:, 512:] / (asum[:, 1:2] + 1e-16)
    h = _elu(0.5 * (ha + hb))
    h = _ln_rows(h, g_ref[...], b_ref[...])
    h = jnp.dot(h, wc_ref[...], preferred_element_type=jnp.float32) + bc_ref[...]
    h = _elu(h)
    res = jnp.dot(x_ref[...], wr_ref[...], preferred_element_type=jnp.float32)
    o_ref[...] = res + br_ref[...] + h


def _tc_post(msum2, asum2, X, g, b, Wc, bc, Wr, br):
    return pl.pallas_call(
        _post_body,
        grid=(N_BLKS,),
        in_specs=[
            pl.BlockSpec((ROWS_BLK, 1024), lambda i: (i, 0)),
            pl.BlockSpec((ROWS_BLK, 2), lambda i: (i, 0)),
            pl.BlockSpec((ROWS_BLK, 512), lambda i: (i, 0)),
            pl.BlockSpec((512,), lambda i: (0,)),
            pl.BlockSpec((512,), lambda i: (0,)),
            pl.BlockSpec((512, 512), lambda i: (0, 0)),
            pl.BlockSpec((512,), lambda i: (0,)),
            pl.BlockSpec((512, 512), lambda i: (0, 0)),
            pl.BlockSpec((512,), lambda i: (0,)),
        ],
        out_specs=pl.BlockSpec((ROWS_BLK, 512), lambda i: (i, 0)),
        out_shape=jax.ShapeDtypeStruct((N, 512), jnp.float32),
    )(msum2, asum2, X, g, b, Wc, bc, Wr, br)


def _extract_body(cw, w_out, col_out, src_out, cnt_out,
                  rowA, rowB, wbuf, colbuf, srcbuf, cntacc, cntv, semA, semB):
    c = lax.axis_index("c")
    s = lax.axis_index("s")
    wid = c * 16 + s
    base = wid * 128
    iota = lax.iota(jnp.int32, 16)
    zf = jnp.zeros((16,), jnp.float32)
    zi = jnp.zeros((16,), jnp.int32)
    ones = zi + 1

    for k in range(NBKT // 16):
        cntacc[pl.ds(k * 16, 16)] = zi

    def zbody(i, _):
        wbuf[pl.ds(i * 16, 16)] = zf
        colbuf[pl.ds(i * 16, 16)] = zi
        srcbuf[pl.ds(i * 16, 16)] = zi
        return 0
    lax.fori_loop(0, NBKT * CAP // 16, zbody, 0)

    bufs = (rowA, rowB)
    sems = (semA, semB)

    def issue(row, par):
        rr = jnp.minimum(row, N - 1)
        return pltpu.async_copy(cw.at[rr], bufs[par], sems[par])

    def wait(par):
        pltpu.make_async_copy(cw.at[0], bufs[par], sems[par]).wait()

    def process(par, row):
        wait(par)
        buf = bufs[par]

        def cbody(g, buf=buf, row=row):
            ws = [buf[pl.ds(g * 64 + u * 16, 16)] for u in range(4)]
            ms = [w > 0.0 for w in ws]
            mo = (ms[0] | ms[1]) | (ms[2] | ms[3])

            @pl.when(plsc.all_reduce_population_count(mo)[0] > 0)
            def _():
                off = plsc.load_gather(cntacc, [zi + g])[0]
                cs = [plsc.all_reduce_population_count(m)[0] for m in ms]
                ctot = (cs[0] + cs[1]) + (cs[2] + cs[3])
                for u in range(4):
                    ou = jnp.minimum(off, CAP - 16)
                    basei = g * CAP + ou
                    plsc.store_compressed(wbuf.at[pl.ds(basei, 16)],
                                          ws[u], mask=ms[u])
                    plsc.store_compressed(colbuf.at[pl.ds(basei, 16)],
                                          (u * 16) + iota, mask=ms[u])
                    plsc.store_compressed(srcbuf.at[pl.ds(basei, 16)],
                                          zi + row, mask=ms[u])
                    off = off + cs[u]
                plsc.addupdate_scatter(cntacc, [zi + g], zi + ctot,
                                       mask=iota == 0)
        plsc.parallel_loop(0, 64, unroll=4)(cbody)

    issue(base, 0)
    issue(base + 1, 1)

    def rbody(k, _):
        row0 = base + 2 * k
        process(0, row0)
        issue(row0 + 2, 0)
        process(1, row0 + 1)
        issue(row0 + 3, 1)
        return 0

    lax.fori_loop(0, 64, rbody, 0)
    wait(0)
    wait(1)

    for k in range(NBKT // 16):
        cntv[pl.ds(k * 16, 16)] = jnp.minimum(cntacc[pl.ds(k * 16, 16)], CAP)
    pltpu.sync_copy(wbuf, w_out.at[pl.ds(wid * NBKT * CAP, NBKT * CAP)])
    pltpu.sync_copy(colbuf, col_out.at[pl.ds(wid * NBKT * CAP, NBKT * CAP)])
    pltpu.sync_copy(srcbuf, src_out.at[pl.ds(wid * NBKT * CAP, NBKT * CAP)])
    pltpu.sync_copy(cntv, cnt_out.at[pl.ds(wid * NBKT, NBKT)])


def _sc_extract(CW):
    mesh = plsc.VectorSubcoreMesh(core_axis_name="c", subcore_axis_name="s")
    return pl.kernel(
        _extract_body,
        out_type=[
            jax.ShapeDtypeStruct((32 * NBKT * CAP,), jnp.float32),
            jax.ShapeDtypeStruct((32 * NBKT * CAP,), jnp.int32),
            jax.ShapeDtypeStruct((32 * NBKT * CAP,), jnp.int32),
            jax.ShapeDtypeStruct((32 * NBKT,), jnp.int32),
        ],
        mesh=mesh,
        compiler_params=pltpu.CompilerParams(needs_layout_passes=False),
        scratch_types=[
            pltpu.VMEM((4096,), jnp.float32),
            pltpu.VMEM((4096,), jnp.float32),
            pltpu.VMEM((NBKT * CAP,), jnp.float32),
            pltpu.VMEM((NBKT * CAP,), jnp.int32),
            pltpu.VMEM((NBKT * CAP,), jnp.int32),
            pltpu.VMEM((NBKT,), jnp.int32),
            pltpu.VMEM((NBKT,), jnp.int32),
            pltpu.SemaphoreType.DMA,
            pltpu.SemaphoreType.DMA,
        ],
    )(CW)


def _gat_body(h_hbm, ssrc_hbm, sdst_hbm, w_hbm, col_hbm, src_hbm, cnt_hbm,
              msum_out, asum_out,
              ssv, cntv, wstg, cstg, sstg, sdvw, rowA, rowB,
              pbuf, asum_w, strip, semA, semB, *, heads):
    wcm, ccm, scm = wstg, cstg, sstg
    F = 1024 // heads
    c = lax.axis_index("c")
    s = lax.axis_index("s")
    T = c * 16 + s
    iota = lax.iota(jnp.int32, 16)
    zf = jnp.zeros((16,), jnp.float32)
    zi = jnp.zeros((16,), jnp.int32)

    pltpu.sync_copy(ssrc_hbm, ssv)
    pltpu.sync_copy(cnt_hbm, cntv)

    bufs = (rowA, rowB)
    sems = (semA, semB)

    for pi in range(2):
        b = 2 * T + pi

        def zs(i, _):
            plsc.store_scatter(strip, [zi + (i // 64), (i % 64) * 16 + iota],
                               zf)
            return 0
        lax.fori_loop(0, BKT * 1024 // 16, zs, 0)

        def za(i, _):
            asum_w[pl.ds(i * 16, 16)] = zf
            return 0
        lax.fori_loop(0, BKT * heads // 16, za, 0)

        pltpu.sync_copy(sdst_hbm.at[pl.ds(b * BKT * heads, BKT * heads)],
                        sdvw)
        descs = []
        for w in range(32):
            src0 = w * NBKT * CAP + b * CAP
            descs.append(pltpu.async_copy(
                w_hbm.at[pl.ds(src0, CAP)], wstg.at[pl.ds(w * CAP, CAP)],
                semA))
            descs.append(pltpu.async_copy(
                col_hbm.at[pl.ds(src0, CAP)], cstg.at[pl.ds(w * CAP, CAP)],
                semA))
            descs.append(pltpu.async_copy(
                src_hbm.at[pl.ds(src0, CAP)], sstg.at[pl.ds(w * CAP, CAP)],
                semA))
        for d in descs:
            d.wait()

        pos = jnp.int32(0)
        for w in range(32):
            cw = plsc.load_gather(cntv, [zi + (w * NBKT + b)])[0]
            nbw = (cw + 15) // 16

            def cpb(bb, pos, w=w, cw=cw):
                lv = bb * 16 + iota
                mv = lv < cw
                pp = jnp.minimum(pos, CCAP - 16)
                vw = wstg[pl.ds(w * CAP + bb * 16, 16)]
                vc = cstg[pl.ds(w * CAP + bb * 16, 16)]
                vs = sstg[pl.ds(w * CAP + bb * 16, 16)]
                plsc.store_compressed(wcm.at[pl.ds(pp, 16)], vw, mask=mv)
                plsc.store_compressed(ccm.at[pl.ds(pp, 16)], vc, mask=mv)
                plsc.store_compressed(scm.at[pl.ds(pp, 16)], vs, mask=mv)
                return pos + jnp.minimum(cw - bb * 16, 16)
            pos = lax.fori_loop(0, nbw, cpb, pos)
        total = jnp.minimum(pos, CCAP - 64)
        for k in range(4):
            tp = total + k * 16
            wcm[pl.ds(tp, 16)] = zf
            ccm[pl.ds(tp, 16)] = zi
            scm[pl.ds(tp, 16)] = zi

        nb = (total + 15) // 16
        nb2 = (nb + 1) // 2

        def bofs(bb):
            return jnp.minimum(bb * 16, CCAP - 16)

        def issue(bb, par):
            sv = scm[pl.ds(bofs(bb), 16)]
            return pltpu.async_copy(h_hbm.at[sv], bufs[par], sems[par])

        def wait(par):
            sv = scm[pl.ds(0, 16)]
            pltpu.make_async_copy(h_hbm.at[sv], bufs[par], sems[par]).wait()

        def compute(bb, par, total=total):
            valid = (bb * 16 + iota) < total
            o = bofs(bb)
            sv = scm[pl.ds(o, 16)]
            cvv = ccm[pl.ds(o, 16)]
            wv16 = wcm[pl.ds(o, 16)]
            svh = sv * heads
            cvh = cvv * heads
            buf = bufs[par]
            for h in range(heads):
                ss = plsc.load_gather(ssv, [svh + h])
                sd = plsc.load_gather(sdvw, [cvh + h])
                z = ss + sd
                z = jnp.where(z > 0, z, 0.2 * z) * wv16
                p = jnp.where(valid, jnp.exp(z), 0.0)
                pbuf[pl.ds(h * 16, 16)] = p
                plsc.addupdate_scatter(asum_w, [cvh + h], p, mask=valid)

            def ebody(e, _):
                ev = zi + e
                dle = plsc.load_gather(ccm, [zi + (o + e)])
                for h in range(heads):
                    pe = plsc.load_gather(pbuf, [zi + (h * 16) + e])

                    def pb(k, ev=ev, dle=dle, pe=pe, h=h):
                        ii = h * F + k * 16 + iota
                        v = plsc.load_gather(buf, [ev, ii])
                        plsc.addupdate_scatter(strip, [dle, ii], v * pe)
                    plsc.parallel_loop(0, F // 16, unroll=16)(pb)
                return 0
            lax.fori_loop(0, 16, ebody, 0)

        issue(0, 0)
        issue(1, 1)

        def tbody(tt, _):
            wait(0)
            compute(2 * tt, 0)
            issue(2 * tt + 2, 0)
            wait(1)
            compute(2 * tt + 1, 1)
            issue(2 * tt + 3, 1)
            return 0
        lax.fori_loop(0, nb2, tbody, 0)
        wait(0)
        wait(1)

        pltpu.sync_copy(strip, msum_out.at[pl.ds(b * BKT, BKT)])
        pltpu.sync_copy(asum_w,
                        asum_out.at[pl.ds(b * BKT * heads, BKT * heads)])


def _sc_gat(h_table, ssrc, sdst, w_e, col_e, src_e, counts, heads):
    mesh = plsc.VectorSubcoreMesh(core_axis_name="c", subcore_axis_name="s")
    body = functools.partial(_gat_body, heads=heads)
    return pl.kernel(
        body,
        out_type=[
            jax.ShapeDtypeStruct((N, 1024), jnp.float32),
            jax.ShapeDtypeStruct((N * heads,), jnp.float32),
        ],
        mesh=mesh,
        compiler_params=pltpu.CompilerParams(needs_layout_passes=False),
        scratch_types=[
            pltpu.VMEM((N * heads,), jnp.float32),
            pltpu.VMEM((32 * NBKT,), jnp.int32),
            pltpu.VMEM((32 * CAP,), jnp.float32),
            pltpu.VMEM((32 * CAP,), jnp.int32),
            pltpu.VMEM((32 * CAP,), jnp.int32),
            pltpu.VMEM((BKT * heads,), jnp.float32),
            pltpu.VMEM((16, 1024), jnp.float32),
            pltpu.VMEM((16, 1024), jnp.float32),
            pltpu.VMEM((16 * heads,), jnp.float32),
            pltpu.VMEM((BKT * heads,), jnp.float32),
            pltpu.VMEM((BKT, 1024), jnp.float32),
            pltpu.SemaphoreType.DMA,
            pltpu.SemaphoreType.DMA,
        ],
    )(h_table, ssrc, sdst, w_e, col_e, src_e, counts)


def _build_attn_mat(a_src, a_dst, heads, f):
    cols = jnp.arange(128)[None, None, :]
    asrc = a_src.reshape(heads, f)[:, :, None]
    adst = a_dst.reshape(heads, f)[:, :, None]
    hh = jnp.arange(heads)[:, None, None]
    A = jnp.where(cols == hh, asrc, 0.0) + jnp.where(cols == 4 + hh, adst, 0.0)
    return A.reshape(heads * f, 128)


def kernel(X, CW, ln_in_g, ln_in_b, W1, a_src1, a_dst1, W2, a_src2, a_dst2,
           ln_h_g, ln_h_b, Wc, bc, Wr, br):
    A1 = _build_attn_mat(a_src1, a_dst1, 4, 256)
    A2 = _build_attn_mat(a_src2, a_dst2, 2, 512)

    w_e, col_e, src_e, counts = _sc_extract(CW)

    h1, sc1 = _tc_pre(X, ln_in_g, ln_in_b, W1, A1)
    msum1, asum1 = _sc_gat(h1, sc1[:, 0:4].reshape(-1),
                           sc1[:, 4:8].reshape(-1),
                           w_e, col_e, src_e, counts, 4)

    h2, sc2 = _tc_mid(msum1, asum1.reshape(N, 4), W2, A2)
    msum2, asum2 = _sc_gat(h2, sc2[:, 0:2].reshape(-1),
                           sc2[:, 4:6].reshape(-1),
                           w_e, col_e, src_e, counts, 2)

    return _tc_post(msum2, asum2.reshape(N, 2), X,
                    ln_h_g, ln_h_b, Wc, bc, Wr, br)


# speedup vs baseline: 1.0635x; 1.0635x over previous
import functools

import jax
import jax.numpy as jnp
from jax import lax
from jax.experimental import pallas as pl
from jax.experimental.pallas import tpu as pltpu
from jax.experimental.pallas import tpu_sc as plsc

N = 4096
ROWS_BLK = 256
N_BLKS = N // ROWS_BLK
NBKT = 64
BKT = N // NBKT
CAP = 96
CCAP = 32 * CAP


def _elu(x):
    return jnp.where(x > 0, x, jnp.exp(x) - 1.0)


def _ln_rows(x, g, b, eps=1e-5):
    m = jnp.mean(x, axis=-1, keepdims=True)
    v = jnp.mean((x - m) ** 2, axis=-1, keepdims=True)
    return (x - m) / jnp.sqrt(v + eps) * g + b


def _pre_body(x_ref, g_ref, b_ref, w1_ref, a_ref, h_ref, s_ref):
    xn = _ln_rows(x_ref[...], g_ref[...], b_ref[...])
    h = jnp.dot(xn, w1_ref[...], preferred_element_type=jnp.float32)
    h_ref[...] = h
    s_ref[...] = jnp.dot(h, a_ref[...], preferred_element_type=jnp.float32)


def _tc_pre(X, g, b, W1, A):
    return pl.pallas_call(
        _pre_body,
        grid=(N_BLKS,),
        in_specs=[
            pl.BlockSpec((ROWS_BLK, 512), lambda i: (i, 0)),
            pl.BlockSpec((512,), lambda i: (0,)),
            pl.BlockSpec((512,), lambda i: (0,)),
            pl.BlockSpec((512, 1024), lambda i: (0, 0)),
            pl.BlockSpec((1024, 128), lambda i: (0, 0)),
        ],
        out_specs=[
            pl.BlockSpec((ROWS_BLK, 1024), lambda i: (i, 0)),
            pl.BlockSpec((ROWS_BLK, 128), lambda i: (i, 0)),
        ],
        out_shape=[
            jax.ShapeDtypeStruct((N, 1024), jnp.float32),
            jax.ShapeDtypeStruct((N, 128), jnp.float32),
        ],
    )(X, g, b, W1, A)


def _mid_body(ms_ref, a_ref, w2_ref, am_ref, h_ref, s_ref, *, heads):
    f = 1024 // heads
    asum = a_ref[...]
    div = jnp.repeat(asum + 1e-16, f, axis=1)
    h1 = _elu(ms_ref[...] / div)
    h = jnp.dot(h1, w2_ref[...], preferred_element_type=jnp.float32)
    h_ref[...] = h
    s_ref[...] = jnp.dot(h, am_ref[...], preferred_element_type=jnp.float32)


def _tc_mid(msum1, asum1, W2, A):
    heads = asum1.shape[-1]
    body = functools.partial(_mid_body, heads=heads)
    return pl.pallas_call(
        body,
        grid=(N_BLKS,),
        in_specs=[
            pl.BlockSpec((ROWS_BLK, 1024), lambda i: (i, 0)),
            pl.BlockSpec((ROWS_BLK, heads), lambda i: (i, 0)),
            pl.BlockSpec((1024, 1024), lambda i: (0, 0)),
            pl.BlockSpec((1024, 128), lambda i: (0, 0)),
        ],
        out_specs=[
            pl.BlockSpec((ROWS_BLK, 1024), lambda i: (i, 0)),
            pl.BlockSpec((ROWS_BLK, 128), lambda i: (i, 0)),
        ],
        out_shape=[
            jax.ShapeDtypeStruct((N, 1024), jnp.float32),
            jax.ShapeDtypeStruct((N, 128), jnp.float32),
        ],
    )(msum1, asum1, W2, A)


def _post_body(ms_ref, a_ref, x_ref, g_ref, b_ref, wc_ref, bc_ref,
               wr_ref, br_ref, o_ref):
    asum = a_ref[...]
    ha = ms_ref[:, :512] / (asum[:, 0:1] + 1e-16)
    hb = ms_ref[:, 512:] / (asum[:, 1:2] + 1e-16)
    h = _elu(0.5 * (ha + hb))
    h = _ln_rows(h, g_ref[...], b_ref[...])
    h = jnp.dot(h, wc_ref[...], preferred_element_type=jnp.float32) + bc_ref[...]
    h = _elu(h)
    res = jnp.dot(x_ref[...], wr_ref[...], preferred_element_type=jnp.float32)
    o_ref[...] = res + br_ref[...] + h


def _tc_post(msum2, asum2, X, g, b, Wc, bc, Wr, br):
    return pl.pallas_call(
        _post_body,
        grid=(N_BLKS,),
        in_specs=[
            pl.BlockSpec((ROWS_BLK, 1024), lambda i: (i, 0)),
            pl.BlockSpec((ROWS_BLK, 2), lambda i: (i, 0)),
            pl.BlockSpec((ROWS_BLK, 512), lambda i: (i, 0)),
            pl.BlockSpec((512,), lambda i: (0,)),
            pl.BlockSpec((512,), lambda i: (0,)),
            pl.BlockSpec((512, 512), lambda i: (0, 0)),
            pl.BlockSpec((512,), lambda i: (0,)),
            pl.BlockSpec((512, 512), lambda i: (0, 0)),
            pl.BlockSpec((512,), lambda i: (0,)),
        ],
        out_specs=pl.BlockSpec((ROWS_BLK, 512), lambda i: (i, 0)),
        out_shape=jax.ShapeDtypeStruct((N, 512), jnp.float32),
    )(msum2, asum2, X, g, b, Wc, bc, Wr, br)


def _extract_body(cw, w_out, col_out, src_out, cnt_out,
                  rowA, rowB, wbuf, colbuf, srcbuf, cntacc, cntv, semA, semB):
    c = lax.axis_index("c")
    s = lax.axis_index("s")
    wid = c * 16 + s
    base = wid * 128
    iota = lax.iota(jnp.int32, 16)
    zf = jnp.zeros((16,), jnp.float32)
    zi = jnp.zeros((16,), jnp.int32)
    ones = zi + 1

    for k in range(NBKT // 16):
        cntacc[pl.ds(k * 16, 16)] = zi

    def zbody(i, _):
        wbuf[pl.ds(i * 16, 16)] = zf
        colbuf[pl.ds(i * 16, 16)] = zi
        srcbuf[pl.ds(i * 16, 16)] = zi
        return 0
    lax.fori_loop(0, NBKT * CAP // 16, zbody, 0)

    bufs = (rowA, rowB)
    sems = (semA, semB)

    def issue(row, par):
        rr = jnp.minimum(row, N - 1)
        return pltpu.async_copy(cw.at[rr], bufs[par], sems[par])

    def wait(par):
        pltpu.make_async_copy(cw.at[0], bufs[par], sems[par]).wait()

    def process(par, row):
        wait(par)
        buf = bufs[par]

        def cbody(g, buf=buf, row=row):
            ws = [buf[pl.ds(g * 64 + u * 16, 16)] for u in range(4)]
            ms = [w > 0.0 for w in ws]
            mo = (ms[0] | ms[1]) | (ms[2] | ms[3])

            @pl.when(plsc.all_reduce_population_count(mo)[0] > 0)
            def _():
                off = plsc.load_gather(cntacc, [zi + g])[0]
                cs = [plsc.all_reduce_population_count(m)[0] for m in ms]
                ctot = (cs[0] + cs[1]) + (cs[2] + cs[3])
                for u in range(4):
                    ou = jnp.minimum(off, CAP - 16)
                    basei = g * CAP + ou
                    plsc.store_compressed(wbuf.at[pl.ds(basei, 16)],
                                          ws[u], mask=ms[u])
                    plsc.store_compressed(colbuf.at[pl.ds(basei, 16)],
                                          (u * 16) + iota, mask=ms[u])
                    plsc.store_compressed(srcbuf.at[pl.ds(basei, 16)],
                                          zi + row, mask=ms[u])
                    off = off + cs[u]
                plsc.addupdate_scatter(cntacc, [zi + g], zi + ctot,
                                       mask=iota == 0)
        plsc.parallel_loop(0, 64, unroll=2)(cbody)

    issue(base, 0)
    issue(base + 1, 1)

    def rbody(k, _):
        row0 = base + 2 * k
        process(0, row0)
        issue(row0 + 2, 0)
        process(1, row0 + 1)
        issue(row0 + 3, 1)
        return 0

    lax.fori_loop(0, 64, rbody, 0)
    wait(0)
    wait(1)

    for k in range(NBKT // 16):
        cntv[pl.ds(k * 16, 16)] = jnp.minimum(cntacc[pl.ds(k * 16, 16)], CAP)
    pltpu.sync_copy(wbuf, w_out.at[pl.ds(wid * NBKT * CAP, NBKT * CAP)])
    pltpu.sync_copy(colbuf, col_out.at[pl.ds(wid * NBKT * CAP, NBKT * CAP)])
    pltpu.sync_copy(srcbuf, src_out.at[pl.ds(wid * NBKT * CAP, NBKT * CAP)])
    pltpu.sync_copy(cntv, cnt_out.at[pl.ds(wid * NBKT, NBKT)])


def _sc_extract(CW):
    mesh = plsc.VectorSubcoreMesh(core_axis_name="c", subcore_axis_name="s")
    return pl.kernel(
        _extract_body,
        out_type=[
            jax.ShapeDtypeStruct((32 * NBKT * CAP,), jnp.float32),
            jax.ShapeDtypeStruct((32 * NBKT * CAP,), jnp.int32),
            jax.ShapeDtypeStruct((32 * NBKT * CAP,), jnp.int32),
            jax.ShapeDtypeStruct((32 * NBKT,), jnp.int32),
        ],
        mesh=mesh,
        compiler_params=pltpu.CompilerParams(needs_layout_passes=False),
        scratch_types=[
            pltpu.VMEM((4096,), jnp.float32),
            pltpu.VMEM((4096,), jnp.float32),
            pltpu.VMEM((NBKT * CAP,), jnp.float32),
            pltpu.VMEM((NBKT * CAP,), jnp.int32),
            pltpu.VMEM((NBKT * CAP,), jnp.int32),
            pltpu.VMEM((NBKT,), jnp.int32),
            pltpu.VMEM((NBKT,), jnp.int32),
            pltpu.SemaphoreType.DMA,
            pltpu.SemaphoreType.DMA,
        ],
    )(CW)


def _gat_body(h_hbm, ssrc_hbm, sdst_hbm, w_hbm, col_hbm, src_hbm, cnt_hbm,
              msum_out, asum_out,
              ssv, cntv, wstg, cstg, sstg, sdvw, rowA, rowB,
              pbuf, asum_w, strip, semA, semB, *, heads):
    wcm, ccm, scm = wstg, cstg, sstg
    F = 1024 // heads
    c = lax.axis_index("c")
    s = lax.axis_index("s")
    T = c * 16 + s
    iota = lax.iota(jnp.int32, 16)
    zf = jnp.zeros((16,), jnp.float32)
    zi = jnp.zeros((16,), jnp.int32)

    pltpu.sync_copy(ssrc_hbm, ssv)
    pltpu.sync_copy(cnt_hbm, cntv)

    bufs = (rowA, rowB)
    sems = (semA, semB)

    for pi in range(2):
        b = 2 * T + pi

        def zs(i, _):
            plsc.store_scatter(strip, [zi + (i // 64), (i % 64) * 16 + iota],
                               zf)
            return 0
        lax.fori_loop(0, BKT * 1024 // 16, zs, 0)

        def za(i, _):
            asum_w[pl.ds(i * 16, 16)] = zf
            return 0
        lax.fori_loop(0, BKT * heads // 16, za, 0)

        pltpu.sync_copy(sdst_hbm.at[pl.ds(b * BKT * heads, BKT * heads)],
                        sdvw)
        descs = []
        for w in range(32):
            src0 = w * NBKT * CAP + b * CAP
            descs.append(pltpu.async_copy(
                w_hbm.at[pl.ds(src0, CAP)], wstg.at[pl.ds(w * CAP, CAP)],
                semA))
            descs.append(pltpu.async_copy(
                col_hbm.at[pl.ds(src0, CAP)], cstg.at[pl.ds(w * CAP, CAP)],
                semA))
            descs.append(pltpu.async_copy(
                src_hbm.at[pl.ds(src0, CAP)], sstg.at[pl.ds(w * CAP, CAP)],
                semA))
        for d in descs:
            d.wait()

        pos = jnp.int32(0)
        for w in range(32):
            cw = plsc.load_gather(cntv, [zi + (w * NBKT + b)])[0]
            nbw = (cw + 15) // 16

            def cpb(bb, pos, w=w, cw=cw):
                lv = bb * 16 + iota
                mv = lv < cw
                pp = jnp.minimum(pos, CCAP - 16)
                vw = wstg[pl.ds(w * CAP + bb * 16, 16)]
                vc = cstg[pl.ds(w * CAP + bb * 16, 16)]
                vs = sstg[pl.ds(w * CAP + bb * 16, 16)]
                plsc.store_compressed(wcm.at[pl.ds(pp, 16)], vw, mask=mv)
                plsc.store_compressed(ccm.at[pl.ds(pp, 16)], vc, mask=mv)
                plsc.store_compressed(scm.at[pl.ds(pp, 16)], vs, mask=mv)
                return pos + jnp.minimum(cw - bb * 16, 16)
            pos = lax.fori_loop(0, nbw, cpb, pos)
        total = jnp.minimum(pos, CCAP - 64)
        for k in range(4):
            tp = total + k * 16
            wcm[pl.ds(tp, 16)] = zf
            ccm[pl.ds(tp, 16)] = zi
            scm[pl.ds(tp, 16)] = zi

        nb = (total + 15) // 16
        nb2 = (nb + 1) // 2

        def bofs(bb):
            return jnp.minimum(bb * 16, CCAP - 16)

        def issue(bb, par):
            sv = scm[pl.ds(bofs(bb), 16)]
            return pltpu.async_copy(h_hbm.at[sv], bufs[par], sems[par])

        def wait(par):
            sv = scm[pl.ds(0, 16)]
            pltpu.make_async_copy(h_hbm.at[sv], bufs[par], sems[par]).wait()

        def compute(bb, par, total=total):
            valid = (bb * 16 + iota) < total
            o = bofs(bb)
            sv = scm[pl.ds(o, 16)]
            cvv = ccm[pl.ds(o, 16)]
            wv16 = wcm[pl.ds(o, 16)]
            svh = sv * heads
            cvh = cvv * heads
            buf = bufs[par]
            for h in range(heads):
                ss = plsc.load_gather(ssv, [svh + h])
                sd = plsc.load_gather(sdvw, [cvh + h])
                z = ss + sd
                z = jnp.where(z > 0, z, 0.2 * z) * wv16
                p = jnp.where(valid, jnp.exp(z), 0.0)
                pbuf[pl.ds(h * 16, 16)] = p
                plsc.addupdate_scatter(asum_w, [cvh + h], p, mask=valid)

            def ebody(e, _):
                ev = zi + e
                dle = plsc.load_gather(ccm, [zi + (o + e)])
                for h in range(heads):
                    pe = plsc.load_gather(pbuf, [zi + (h * 16) + e])

                    def pb(k, ev=ev, dle=dle, pe=pe, h=h):
                        ii = h * F + k * 16 + iota
                        v = plsc.load_gather(buf, [ev, ii])
                        plsc.addupdate_scatter(strip, [dle, ii], v * pe)
                    plsc.parallel_loop(0, F // 16, unroll=16)(pb)
                return 0
            lax.fori_loop(0, 16, ebody, 0)

        issue(0, 0)
        issue(1, 1)

        def tbody(tt, _):
            wait(0)
            compute(2 * tt, 0)
            issue(2 * tt + 2, 0)
            wait(1)
            compute(2 * tt + 1, 1)
            issue(2 * tt + 3, 1)
            return 0
        lax.fori_loop(0, nb2, tbody, 0)
        wait(0)
        wait(1)

        pltpu.sync_copy(strip, msum_out.at[pl.ds(b * BKT, BKT)])
        pltpu.sync_copy(asum_w,
                        asum_out.at[pl.ds(b * BKT * heads, BKT * heads)])


def _sc_gat(h_table, ssrc, sdst, w_e, col_e, src_e, counts, heads):
    mesh = plsc.VectorSubcoreMesh(core_axis_name="c", subcore_axis_name="s")
    body = functools.partial(_gat_body, heads=heads)
    return pl.kernel(
        body,
        out_type=[
            jax.ShapeDtypeStruct((N, 1024), jnp.float32),
            jax.ShapeDtypeStruct((N * heads,), jnp.float32),
        ],
        mesh=mesh,
        compiler_params=pltpu.CompilerParams(needs_layout_passes=False),
        scratch_types=[
            pltpu.VMEM((N * heads,), jnp.float32),
            pltpu.VMEM((32 * NBKT,), jnp.int32),
            pltpu.VMEM((32 * CAP,), jnp.float32),
            pltpu.VMEM((32 * CAP,), jnp.int32),
            pltpu.VMEM((32 * CAP,), jnp.int32),
            pltpu.VMEM((BKT * heads,), jnp.float32),
            pltpu.VMEM((16, 1024), jnp.float32),
            pltpu.VMEM((16, 1024), jnp.float32),
            pltpu.VMEM((16 * heads,), jnp.float32),
            pltpu.VMEM((BKT * heads,), jnp.float32),
            pltpu.VMEM((BKT, 1024), jnp.float32),
            pltpu.SemaphoreType.DMA,
            pltpu.SemaphoreType.DMA,
        ],
    )(h_table, ssrc, sdst, w_e, col_e, src_e, counts)


def _build_attn_mat(a_src, a_dst, heads, f):
    cols = jnp.arange(128)[None, None, :]
    asrc = a_src.reshape(heads, f)[:, :, None]
    adst = a_dst.reshape(heads, f)[:, :, None]
    hh = jnp.arange(heads)[:, None, None]
    A = jnp.where(cols == hh, asrc, 0.0) + jnp.where(cols == 4 + hh, adst, 0.0)
    return A.reshape(heads * f, 128)


def kernel(X, CW, ln_in_g, ln_in_b, W1, a_src1, a_dst1, W2, a_src2, a_dst2,
           ln_h_g, ln_h_b, Wc, bc, Wr, br):
    A1 = _build_attn_mat(a_src1, a_dst1, 4, 256)
    A2 = _build_attn_mat(a_src2, a_dst2, 2, 512)

    w_e, col_e, src_e, counts = _sc_extract(CW)

    h1, sc1 = _tc_pre(X, ln_in_g, ln_in_b, W1, A1)
    msum1, asum1 = _sc_gat(h1, sc1[:, 0:4].reshape(-1),
                           sc1[:, 4:8].reshape(-1),
                           w_e, col_e, src_e, counts, 4)

    h2, sc2 = _tc_mid(msum1, asum1.reshape(N, 4), W2, A2)
    msum2, asum2 = _sc_gat(h2, sc2[:, 0:2].reshape(-1),
                           sc2[:, 4:6].reshape(-1),
                           w_e, col_e, src_e, counts, 2)

    return _tc_post(msum2, asum2.reshape(N, 2), X,
                    ln_h_g, ln_h_b, Wc, bc, Wr, br)
